# Initial kernel scaffold; baseline (speedup 1.0000x reference)
import jax
import jax.numpy as jnp
from jax.experimental import pallas as pl


def kernel(genres, table):
    raise NotImplementedError("write your pallas kernel here")


# speedup vs baseline: 3.7023x; 3.7023x over previous
import functools

import jax
import jax.numpy as jnp
from jax import lax
from jax.experimental import pallas as pl
from jax.experimental.pallas import tpu as pltpu
from jax.experimental.pallas import tpu_sc as plsc

NUM_ROWS = 129
EMBED_D = 64
TOTAL_B = 4096 * 200

_NC = 2
_NS = 16
_NW = _NC * _NS
_BPW = TOTAL_B // _NW
_CHUNK = 128
_NCHUNK = _BPW // _CHUNK
_L = 16

_mesh = plsc.VectorSubcoreMesh(core_axis_name="c", subcore_axis_name="s")


@functools.partial(
    pl.kernel,
    mesh=_mesh,
    out_type=jax.ShapeDtypeStruct((TOTAL_B, EMBED_D), jnp.float32),
    scratch_types=[
        pltpu.VMEM((NUM_ROWS, EMBED_D), jnp.float32),
        pltpu.VMEM((_CHUNK,), jnp.int32),
        pltpu.VMEM((_CHUNK, EMBED_D), jnp.float32),
        pltpu.SemaphoreType.DMA,
    ],
)
def _embed_gather(idx_hbm, table_hbm, out_hbm, table_v, idx_v, rows_v, sem):
    wid = lax.axis_index("s") * _NC + lax.axis_index("c")
    base = wid * _BPW

    pltpu.sync_copy(table_hbm, table_v)

    def body(c, carry):
        off = base + c * _CHUNK
        pltpu.sync_copy(idx_hbm.at[pl.ds(off, _CHUNK)], idx_v)
        for g in range(_CHUNK // _L):
            iv = idx_v[pl.ds(g * _L, _L)]
            for r in range(_L):
                row = iv[r]
                for j in range(EMBED_D // _L):
                    rows_v[g * _L + r, pl.ds(j * _L, _L)] = (
                        table_v[row, pl.ds(j * _L, _L)])
        pltpu.sync_copy(rows_v, out_hbm.at[pl.ds(off, _CHUNK)])
        return carry

    lax.fori_loop(0, _NCHUNK, body, 0)


def kernel(genres, table):
    flat = genres.reshape(-1).astype(jnp.int32)
    out = _embed_gather(flat, table)
    return out.reshape(genres.shape + (EMBED_D,))


# speedup vs baseline: 5.1406x; 1.3885x over previous
import functools

import jax
import jax.numpy as jnp
from jax import lax
from jax.experimental import pallas as pl
from jax.experimental.pallas import tpu as pltpu
from jax.experimental.pallas import tpu_sc as plsc

NUM_ROWS = 129
EMBED_D = 64
TOTAL_B = 4096 * 200

_NC = 2
_NS = 16
_NW = _NC * _NS
_BPW = TOTAL_B // _NW
_CHUNK = 160
_NCHUNK = _BPW // _CHUNK
_L = 16

_mesh = plsc.VectorSubcoreMesh(core_axis_name="c", subcore_axis_name="s")


@functools.partial(
    pl.kernel,
    mesh=_mesh,
    out_type=jax.ShapeDtypeStruct((TOTAL_B, EMBED_D), jnp.float32),
    scratch_types=[
        pltpu.VMEM((NUM_ROWS, EMBED_D), jnp.float32),
        pltpu.VMEM((_BPW,), jnp.int32),
        pltpu.VMEM((_CHUNK, EMBED_D), jnp.float32),
        pltpu.VMEM((_CHUNK, EMBED_D), jnp.float32),
        pltpu.SemaphoreType.DMA,
        pltpu.SemaphoreType.DMA,
    ],
)
def _embed_gather(idx_hbm, table_hbm, out_hbm, table_v, idx_all,
                  rows0, rows1, sem0, sem1):
    wid = lax.axis_index("s") * _NC + lax.axis_index("c")
    base = wid * _BPW

    pltpu.sync_copy(table_hbm, table_v)
    pltpu.sync_copy(idx_hbm.at[pl.ds(base, _BPW)], idx_all)

    def expand(c, rv):
        def grp(g, carry):
            iv = idx_all[pl.ds(c * _CHUNK + g * _L, _L)]
            for r in range(_L):
                row = iv[r]
                dst = g * _L + r
                for j in range(EMBED_D // _L):
                    rv[dst, pl.ds(j * _L, _L)] = table_v[row,
                                                         pl.ds(j * _L, _L)]
            return carry
        lax.fori_loop(0, _CHUNK // _L, grp, 0)

    def pair(i, carry):
        for b, (rv, sem) in enumerate(((rows0, sem0), (rows1, sem1))):
            c = 2 * i + b

            @pl.when(i > 0)
            def _wait_prev():
                pltpu.make_async_copy(
                    rv, out_hbm.at[pl.ds(base, _CHUNK)], sem).wait()

            expand(c, rv)
            pltpu.async_copy(
                rv, out_hbm.at[pl.ds(base + c * _CHUNK, _CHUNK)], sem)
        return carry

    lax.fori_loop(0, _NCHUNK // 2, pair, 0)

    pltpu.make_async_copy(rows0, out_hbm.at[pl.ds(base, _CHUNK)], sem0).wait()
    pltpu.make_async_copy(rows1, out_hbm.at[pl.ds(base, _CHUNK)], sem1).wait()


def kernel(genres, table):
    flat = genres.reshape(-1).astype(jnp.int32)
    out = _embed_gather(flat, table)
    return out.reshape(genres.shape + (EMBED_D,))
